# Initial kernel scaffold; baseline (speedup 1.0000x reference)
import jax
import jax.numpy as jnp
from jax.experimental import pallas as pl


def kernel(v, eidx, esgn):
    raise NotImplementedError("write your pallas kernel here")


# speedup vs baseline: 3.0466x; 3.0466x over previous
import functools

import jax
import jax.numpy as jnp
from jax import lax
from jax.experimental import pallas as pl
from jax.experimental.pallas import tpu as pltpu
from jax.experimental.pallas import tpu_sc as plsc


def _make_sc_kernel(n_edges, d, chunk, n_chunks, epw):
    mesh = plsc.VectorSubcoreMesh(core_axis_name="c", subcore_axis_name="s")

    @functools.partial(
        pl.kernel,
        mesh=mesh,
        out_type=jax.ShapeDtypeStruct((n_edges,), jnp.float32),
        compiler_params=pltpu.CompilerParams(needs_layout_passes=False),
        scratch_types=[
            pltpu.VMEM((chunk,), jnp.int32),
            pltpu.VMEM((chunk,), jnp.int32),
            pltpu.VMEM((chunk,), jnp.float32),
            pltpu.VMEM((chunk, d), jnp.float32),
            pltpu.VMEM((chunk, d), jnp.float32),
            pltpu.VMEM((chunk,), jnp.float32),
            pltpu.SemaphoreType.DMA,
        ],
    )
    def sc_kernel(v_hbm, s_hbm, t_hbm, g_hbm, out_hbm,
                  si_v, ti_v, g_v, rs_v, rt_v, o_v, sem):
        wid = lax.axis_index("s") * 2 + lax.axis_index("c")
        base = wid * epw

        def chunk_body(ci, carry):
            cb = base + ci * chunk
            pltpu.sync_copy(s_hbm.at[pl.ds(cb, chunk)], si_v)
            pltpu.sync_copy(t_hbm.at[pl.ds(cb, chunk)], ti_v)
            pltpu.sync_copy(g_hbm.at[pl.ds(cb, chunk)], g_v)
            pltpu.async_copy(v_hbm.at[si_v], rs_v, sem).wait()
            pltpu.async_copy(v_hbm.at[ti_v], rt_v, sem).wait()

            lanes = lax.iota(jnp.int32, 16)

            def group_body(g, c2):
                e0 = g * 16
                res = jnp.zeros((16,), jnp.float32)
                for j in range(16):
                    e = e0 + j
                    acc = rs_v[e, pl.ds(0, 16)] * rt_v[e, pl.ds(0, 16)]
                    for k in range(1, d // 16):
                        acc = acc + (rs_v[e, pl.ds(k * 16, 16)]
                                     * rt_v[e, pl.ds(k * 16, 16)])
                    res = jnp.where(lanes == j, jnp.sum(acc), res)
                o_v[pl.ds(e0, 16)] = res * g_v[pl.ds(e0, 16)]
                return c2

            lax.fori_loop(0, chunk // 16, group_body, 0)
            pltpu.sync_copy(o_v, out_hbm.at[pl.ds(cb, chunk)])
            return carry

        lax.fori_loop(0, n_chunks, chunk_body, 0)

    return sc_kernel


def kernel(v, eidx, esgn):
    n_nodes, d = v.shape
    n_edges = esgn.shape[0]
    n_workers = 32
    epw = n_edges // n_workers
    chunk = 400
    n_chunks = epw // chunk
    assert epw * n_workers == n_edges and n_chunks * chunk == epw

    sidx = eidx[0].astype(jnp.int32)
    tidx = eidx[1].astype(jnp.int32)
    sc = _make_sc_kernel(n_edges, d, chunk, n_chunks, epw)
    return sc(v, sidx, tidx, esgn)


# speedup vs baseline: 4.0460x; 1.3280x over previous
import functools

import jax
import jax.numpy as jnp
from jax import lax
from jax.experimental import pallas as pl
from jax.experimental.pallas import tpu as pltpu
from jax.experimental.pallas import tpu_sc as plsc


def _make_sc_kernel(n_edges, d, chunk, n_chunks, epw):
    mesh = plsc.VectorSubcoreMesh(core_axis_name="c", subcore_axis_name="s")

    @functools.partial(
        pl.kernel,
        mesh=mesh,
        out_type=jax.ShapeDtypeStruct((n_edges,), jnp.float32),
        compiler_params=pltpu.CompilerParams(needs_layout_passes=False),
        scratch_types=[
            pltpu.VMEM((epw,), jnp.int32),
            pltpu.VMEM((epw,), jnp.int32),
            pltpu.VMEM((epw,), jnp.float32),
            pltpu.VMEM((epw,), jnp.float32),
            pltpu.VMEM((chunk, d), jnp.float32),
            pltpu.VMEM((chunk, d), jnp.float32),
            pltpu.VMEM((chunk, d), jnp.float32),
            pltpu.VMEM((chunk, d), jnp.float32),
            pltpu.SemaphoreType.DMA,
            pltpu.SemaphoreType.DMA,
        ],
    )
    def sc_kernel(v_hbm, s_hbm, t_hbm, g_hbm, out_hbm,
                  si_v, ti_v, g_v, o_v, rs0, rt0, rs1, rt1, sem0, sem1):
        wid = lax.axis_index("s") * 2 + lax.axis_index("c")
        base = wid * epw
        pltpu.sync_copy(s_hbm.at[pl.ds(base, epw)], si_v)
        pltpu.sync_copy(t_hbm.at[pl.ds(base, epw)], ti_v)
        pltpu.sync_copy(g_hbm.at[pl.ds(base, epw)], g_v)

        rbufs = ((rs0, rt0, sem0), (rs1, rt1, sem1))
        lanes = lax.iota(jnp.int32, 16)

        def gathers(ci, b):
            rs, rt, sem = rbufs[b]
            sl = pl.ds(ci * chunk, chunk)
            return (pltpu.make_async_copy(v_hbm.at[si_v.at[sl]], rs, sem),
                    pltpu.make_async_copy(v_hbm.at[ti_v.at[sl]], rt, sem))

        def issue(ci, b):
            ds, dt = gathers(ci, b)
            ds.start()
            dt.start()

        def compute(ci, b):
            rs, rt, _ = rbufs[b]
            cb = ci * chunk

            def group_body(g, c2):
                e0 = g * 16
                res = jnp.zeros((16,), jnp.float32)
                for j in range(16):
                    e = e0 + j
                    acc = rs[e, pl.ds(0, 16)] * rt[e, pl.ds(0, 16)]
                    for k in range(1, d // 16):
                        acc = acc + (rs[e, pl.ds(k * 16, 16)]
                                     * rt[e, pl.ds(k * 16, 16)])
                    res = jnp.where(lanes == j, jnp.sum(acc), res)
                o_v[pl.ds(cb + e0, 16)] = res * g_v[pl.ds(cb + e0, 16)]
                return c2

            lax.fori_loop(0, chunk // 16, group_body, 0)

        def step(ci, b, issue_next):
            ds, dt = gathers(ci, b)
            ds.wait()
            dt.wait()
            compute(ci, b)
            if issue_next:
                issue(ci + 2, b)

        issue(0, 0)
        issue(1, 1)
        main_pairs = (n_chunks - 2) // 2

        @pl.loop(0, main_pairs)
        def _pair(it):
            ci0 = it * 2
            step(ci0, 0, True)
            step(ci0 + 1, 1, True)

        for ci in range(2 * main_pairs, n_chunks):
            step(ci, ci % 2, ci + 2 < n_chunks)

        pltpu.sync_copy(o_v, out_hbm.at[pl.ds(base, epw)])

    return sc_kernel


def kernel(v, eidx, esgn):
    n_nodes, d = v.shape
    n_edges = esgn.shape[0]
    n_workers = 32
    epw = n_edges // n_workers
    chunk = 80
    n_chunks = epw // chunk
    assert epw * n_workers == n_edges and n_chunks * chunk == epw

    sidx = eidx[0].astype(jnp.int32)
    tidx = eidx[1].astype(jnp.int32)
    sc = _make_sc_kernel(n_edges, d, chunk, n_chunks, epw)
    return sc(v, sidx, tidx, esgn)


# speedup vs baseline: 9.8624x; 2.4376x over previous
import functools

import jax
import jax.numpy as jnp
from jax import lax
from jax.experimental import pallas as pl
from jax.experimental.pallas import tpu as pltpu
from jax.experimental.pallas import tpu_sc as plsc


def _make_sc_kernel(n_edges, d, chunk, n_chunks, epw):
    mesh = plsc.VectorSubcoreMesh(core_axis_name="c", subcore_axis_name="s")

    @functools.partial(
        pl.kernel,
        mesh=mesh,
        out_type=jax.ShapeDtypeStruct((n_edges,), jnp.float32),
        compiler_params=pltpu.CompilerParams(
            needs_layout_passes=False, use_tc_tiling_on_sc=False),
        scratch_types=[
            pltpu.VMEM((epw,), jnp.int32),
            pltpu.VMEM((epw,), jnp.int32),
            pltpu.VMEM((epw,), jnp.float32),
            pltpu.VMEM((epw,), jnp.float32),
            pltpu.VMEM((chunk, d // 2), jnp.int32),
            pltpu.VMEM((chunk, d // 2), jnp.int32),
            pltpu.VMEM((chunk, d // 2), jnp.int32),
            pltpu.VMEM((chunk, d // 2), jnp.int32),
            pltpu.SemaphoreType.DMA,
            pltpu.SemaphoreType.DMA,
        ],
    )
    def sc_kernel(v_hbm, s_hbm, t_hbm, g_hbm, out_hbm,
                  si_v, ti_v, g_v, o_v, rs0, rt0, rs1, rt1, sem0, sem1):
        wid = lax.axis_index("s") * 2 + lax.axis_index("c")
        base = wid * epw
        pltpu.sync_copy(s_hbm.at[pl.ds(base, epw)], si_v)
        pltpu.sync_copy(t_hbm.at[pl.ds(base, epw)], ti_v)
        pltpu.sync_copy(g_hbm.at[pl.ds(base, epw)], g_v)

        rbufs = ((rs0, rt0, sem0), (rs1, rt1, sem1))
        lanes = lax.iota(jnp.int32, 16)

        def gathers(ci, b):
            rs, rt, sem = rbufs[b]
            sl = pl.ds(ci * chunk, chunk)
            return (pltpu.make_async_copy(v_hbm.at[si_v.at[sl]], rs, sem),
                    pltpu.make_async_copy(v_hbm.at[ti_v.at[sl]], rt, sem))

        def issue(ci, b):
            ds, dt = gathers(ci, b)
            ds.start()
            dt.start()

        def compute(ci, b):
            rs, rt, _ = rbufs[b]
            cb = ci * chunk

            def group_body(g, c2):
                e0 = g * 16
                res = jnp.zeros((16,), jnp.float32)
                for j in range(16):
                    e = e0 + j
                    acc = jnp.zeros((16,), jnp.float32)
                    for k in range(d // 32):
                        a = plsc.bitcast(rs[e, pl.ds(k * 16, 16)],
                                         jnp.bfloat16)
                        b = plsc.bitcast(rt[e, pl.ds(k * 16, 16)],
                                         jnp.bfloat16)
                        p = a * b
                        p0, p1 = plsc.unpack(
                            p, format=plsc.PackFormat.INTERLEAVED)
                        acc = acc + p0 + p1
                    res = jnp.where(lanes == j, jnp.sum(acc), res)
                o_v[pl.ds(cb + e0, 16)] = res * g_v[pl.ds(cb + e0, 16)]
                return c2

            lax.fori_loop(0, chunk // 16, group_body, 0)

        def step(ci, b, issue_next):
            ds, dt = gathers(ci, b)
            ds.wait()
            dt.wait()
            compute(ci, b)
            if issue_next:
                issue(ci + 2, b)

        issue(0, 0)
        issue(1, 1)
        main_pairs = (n_chunks - 2) // 2

        @pl.loop(0, main_pairs)
        def _pair(it):
            ci0 = it * 2
            step(ci0, 0, True)
            step(ci0 + 1, 1, True)

        for ci in range(2 * main_pairs, n_chunks):
            step(ci, ci % 2, ci + 2 < n_chunks)

        pltpu.sync_copy(o_v, out_hbm.at[pl.ds(base, epw)])

    return sc_kernel


def kernel(v, eidx, esgn):
    n_nodes, d = v.shape
    n_edges = esgn.shape[0]
    n_workers = 32
    epw = n_edges // n_workers
    chunk = 80
    n_chunks = epw // chunk
    assert epw * n_workers == n_edges and n_chunks * chunk == epw

    sidx = eidx[0].astype(jnp.int32)
    tidx = eidx[1].astype(jnp.int32)
    v_bf = v.astype(jnp.bfloat16)
    v32 = lax.bitcast_convert_type(
        v_bf.reshape(n_nodes, d // 2, 2), jnp.int32)
    sc = _make_sc_kernel(n_edges, d, chunk, n_chunks, epw)
    return sc(v32, sidx, tidx, esgn)


# speedup vs baseline: 12.0814x; 1.2250x over previous
import functools

import jax
import jax.numpy as jnp
from jax import lax
from jax.experimental import pallas as pl
from jax.experimental.pallas import tpu as pltpu
from jax.experimental.pallas import tpu_sc as plsc


def _make_sc_kernel(n_nodes, n_edges, d, chunk, n_chunks, epw):
    mesh = plsc.VectorSubcoreMesh(core_axis_name="c", subcore_axis_name="s")

    @functools.partial(
        pl.kernel,
        mesh=mesh,
        out_type=jax.ShapeDtypeStruct((n_edges,), jnp.float32),
        compiler_params=pltpu.CompilerParams(
            needs_layout_passes=False, use_tc_tiling_on_sc=False),
        scratch_types=[
            pltpu.VMEM((epw,), jnp.int32),
            pltpu.VMEM((epw,), jnp.int32),
            pltpu.VMEM((epw,), jnp.float32),
            pltpu.VMEM((epw,), jnp.float32),
            pltpu.VMEM((chunk, d // 2), jnp.int32),
            pltpu.VMEM((chunk, d // 2), jnp.int32),
            pltpu.VMEM((chunk, d // 2), jnp.int32),
            pltpu.VMEM((chunk, d // 2), jnp.int32),
            pltpu.SemaphoreType.DMA,
            pltpu.SemaphoreType.DMA,
            pltpu.VMEM_SHARED((n_nodes, d // 2), jnp.int32),
        ],
    )
    def sc_kernel(v_hbm, s_hbm, t_hbm, g_hbm, out_hbm,
                  si_v, ti_v, g_v, o_v, rs0, rt0, rs1, rt1, sem0, sem1,
                  v_sp):
        wid = lax.axis_index("s") * 2 + lax.axis_index("c")
        base = wid * epw

        @pl.when(lax.axis_index("s") == 0)
        def _stage():
            pltpu.sync_copy(v_hbm, v_sp)

        pltpu.sync_copy(s_hbm.at[pl.ds(base, epw)], si_v)
        pltpu.sync_copy(t_hbm.at[pl.ds(base, epw)], ti_v)
        pltpu.sync_copy(g_hbm.at[pl.ds(base, epw)], g_v)
        plsc.subcore_barrier()

        rbufs = ((rs0, rt0, sem0), (rs1, rt1, sem1))
        lanes = lax.iota(jnp.int32, 16)

        def gathers(ci, b):
            rs, rt, sem = rbufs[b]
            sl = pl.ds(ci * chunk, chunk)
            return (pltpu.make_async_copy(v_sp.at[si_v.at[sl]], rs, sem),
                    pltpu.make_async_copy(v_sp.at[ti_v.at[sl]], rt, sem))

        def issue(ci, b):
            ds, dt = gathers(ci, b)
            ds.start()
            dt.start()

        def compute(ci, b):
            rs, rt, _ = rbufs[b]
            cb = ci * chunk

            def group_body(g, c2):
                e0 = g * 16
                res = jnp.zeros((16,), jnp.float32)
                for j in range(16):
                    e = e0 + j
                    acc = jnp.zeros((16,), jnp.float32)
                    for k in range(d // 32):
                        a = plsc.bitcast(rs[e, pl.ds(k * 16, 16)],
                                         jnp.bfloat16)
                        b = plsc.bitcast(rt[e, pl.ds(k * 16, 16)],
                                         jnp.bfloat16)
                        p = a * b
                        p0, p1 = plsc.unpack(
                            p, format=plsc.PackFormat.INTERLEAVED)
                        acc = acc + p0 + p1
                    res = jnp.where(lanes == j, jnp.sum(acc), res)
                o_v[pl.ds(cb + e0, 16)] = res * g_v[pl.ds(cb + e0, 16)]
                return c2

            lax.fori_loop(0, chunk // 16, group_body, 0)

        def step(ci, b, issue_next):
            ds, dt = gathers(ci, b)
            ds.wait()
            dt.wait()
            compute(ci, b)
            if issue_next:
                issue(ci + 2, b)

        issue(0, 0)
        issue(1, 1)
        main_pairs = (n_chunks - 2) // 2

        @pl.loop(0, main_pairs)
        def _pair(it):
            ci0 = it * 2
            step(ci0, 0, True)
            step(ci0 + 1, 1, True)

        for ci in range(2 * main_pairs, n_chunks):
            step(ci, ci % 2, ci + 2 < n_chunks)

        pltpu.sync_copy(o_v, out_hbm.at[pl.ds(base, epw)])

    return sc_kernel


def kernel(v, eidx, esgn):
    n_nodes, d = v.shape
    n_edges = esgn.shape[0]
    n_workers = 32
    epw = n_edges // n_workers
    chunk = 80
    n_chunks = epw // chunk
    assert epw * n_workers == n_edges and n_chunks * chunk == epw

    sidx = eidx[0].astype(jnp.int32)
    tidx = eidx[1].astype(jnp.int32)
    v_bf = v.astype(jnp.bfloat16)
    v32 = lax.bitcast_convert_type(
        v_bf.reshape(n_nodes, d // 2, 2), jnp.int32)
    sc = _make_sc_kernel(n_nodes, n_edges, d, chunk, n_chunks, epw)
    return sc(v32, sidx, tidx, esgn)
